# Initial kernel scaffold; baseline (speedup 1.0000x reference)
import jax
import jax.numpy as jnp
from jax.experimental import pallas as pl


def kernel(x, params):
    raise NotImplementedError("write your pallas kernel here")


# speedup vs baseline: 12.3538x; 12.3538x over previous
import functools

import jax
import jax.numpy as jnp
import numpy as np
from jax.experimental import pallas as pl
from jax.experimental.pallas import tpu as pltpu

TOPK1, TOPK2, RAD = 500, 300, 9
NSRC = 512


def _gat_kernel(xn_ref, s_ref, band_ref, wl_ref, bl_ref, wr_ref, br_ref,
                att_ref, wres_ref, bias_ref, out_ref, e_s, xr_s):
    f32 = jnp.float32
    xn = xn_ref[0]
    xl = jnp.dot(xn, wl_ref[...], preferred_element_type=f32) + bl_ref[...]
    xr = jnp.dot(xn, wr_ref[...], preferred_element_type=f32) + br_ref[...]
    xr_s[...] = xr
    sel = s_ref[0]
    xls = jnp.dot(sel, xl, preferred_element_type=f32)
    att = att_ref[...]

    n = xn.shape[0]
    td = 8

    def body(t, carry):
        xr_t = xr_s[pl.ds(t * td, td), :]
        v = xr_t[:, None, :] + xls[None, :, :]
        v = jnp.where(v >= 0, v, 0.2 * v) * att[None, :, :]
        e_s[pl.ds(t * td, td), :] = jnp.sum(v, axis=2)
        return carry

    jax.lax.fori_loop(0, n // td, body, 0)

    band = band_ref[0]
    e = jnp.where(band > 0.5, e_s[...], -1e9)

    vs = xl + xr
    vs = jnp.where(vs >= 0, vs, 0.2 * vs) * att
    sl = jnp.sum(vs, axis=1, keepdims=True)

    m = jnp.maximum(jnp.max(e, axis=1, keepdims=True), sl)
    ex = jnp.where(band > 0.5, jnp.exp(e - m), 0.0)
    exs = jnp.exp(sl - m)
    den = jnp.sum(ex, axis=1, keepdims=True) + exs
    inv = 1.0 / (den + 1e-16)
    out = (jnp.dot(ex * inv, xls, preferred_element_type=f32)
           + (exs * inv) * xl
           + jnp.dot(xn, wres_ref[...], preferred_element_type=f32)
           + bias_ref[...])
    out_ref[0] = jnp.maximum(out, 0.0)


def _gat_layer(xn, sel, band, wl, bl, wr, br, att, wres, bias):
    b, n, din = xn.shape
    wspec = lambda shp: pl.BlockSpec(shp, lambda i: (0,) * len(shp))
    bspec = lambda shp: pl.BlockSpec(shp, lambda i: (i,) + (0,) * (len(shp) - 1))
    return pl.pallas_call(
        _gat_kernel,
        grid=(b,),
        in_specs=[
            bspec((1, n, din)),
            bspec((1, NSRC, n)),
            bspec((1, n, NSRC)),
            wspec((din, 256)), wspec((1, 256)),
            wspec((din, 256)), wspec((1, 256)),
            wspec((1, 256)),
            wspec((din, 256)), wspec((1, 256)),
        ],
        out_specs=bspec((1, n, 256)),
        out_shape=jax.ShapeDtypeStruct((b, n, 256), jnp.float32),
        scratch_shapes=[pltpu.VMEM((n, NSRC), jnp.float32),
                        pltpu.VMEM((n, 256), jnp.float32)],
    )(xn, sel, band,
      wl.T, bl.reshape(1, 256), wr.T, br.reshape(1, 256),
      att.reshape(1, 256), wres.T, bias.reshape(1, 256))


def _pool_matrix(n, m):
    p = np.zeros((n, m), np.float32)
    for i in range(m):
        s = (i * n) // m
        e = -(-((i + 1) * n) // m)
        p[s:e, i] = 1.0 / (e - s)
    return jnp.asarray(p)


def _conv(x, w, stride, pad):
    return jax.lax.conv_general_dilated(
        x, w, (stride, stride), [(pad, pad), (pad, pad)],
        dimension_numbers=('NCHW', 'OIHW', 'NCHW'))


def _group_norm(x, groups, g, b, eps=1e-5):
    bb, c, h, w = x.shape
    xr = x.reshape(bb, groups, c // groups, h, w)
    m = xr.mean(axis=(2, 3, 4), keepdims=True)
    v = xr.var(axis=(2, 3, 4), keepdims=True)
    x = ((xr - m) / jnp.sqrt(v + eps)).reshape(bb, c, h, w)
    return x * g.reshape(1, c, 1, 1) + b.reshape(1, c, 1, 1)


def _pyramid_cnn(x, p):
    r = jax.nn.relu
    x = r(_group_norm(_conv(x, p['conv1_w'], 1, 2), 4, p['gn1_g'], p['gn1_b']))
    x = r(_group_norm(_conv(x, p['conv1_1_w'], 1, 1), 4, p['gn1_1_g'], p['gn1_1_b']))
    x = r(_group_norm(_conv(x, p['conv2_w'], 2, 2), 8, p['gn2_g'], p['gn2_b']))
    x = r(_group_norm(_conv(x, p['conv2_1_w'], 1, 1), 8, p['gn2_1_g'], p['gn2_1_b']))
    x = r(_group_norm(_conv(x, p['conv3_w'], 2, 2), 16, p['gn3_g'], p['gn3_b']))
    x = r(_group_norm(_conv(x, p['conv3_1_w'], 1, 1), 16, p['gn3_1_g'], p['gn3_1_b']))
    ph = _pool_matrix(x.shape[2], 8)
    pw = _pool_matrix(x.shape[3], 112)
    x = jnp.einsum('bchw,hp,wq->bcpq', x, ph, pw)
    bb, c, h, w = x.shape
    return x.reshape(bb, c, h * w).transpose(0, 2, 1)


def _l2norm(x, axis):
    return x / jnp.clip(jnp.linalg.norm(x, axis=axis, keepdims=True), 1e-12)


def _net_vlad(x, p):
    b = x.shape[0]
    x = x.transpose(0, 3, 2, 1).reshape(b, TOPK2, 256)
    act = jax.nn.softmax(x @ p['cluster_w'] + p['cluster_b'], axis=-1)
    a = act.sum(axis=1, keepdims=True) * p['cluster_w2']
    vlad = jnp.matmul(act.transpose(0, 2, 1), x).transpose(0, 2, 1) - a
    vlad = _l2norm(vlad, 1).reshape(b, -1)
    vlad = _l2norm(vlad, 1) @ p['hidden1_w']
    vlad = vlad * p['bn2_g'] + p['bn2_b']
    gates = jax.nn.sigmoid(vlad @ p['gating_w'] + p['gating_b'])
    return vlad * gates


def kernel(x, params):
    p = params
    pf = _pyramid_cnn(x, p)
    scores1 = jnp.linalg.norm(pf, axis=2)
    _, top1 = jax.lax.top_k(scores1, TOPK1)
    h, w = x.shape[2] // 8, x.shape[3] // 8
    n = h * w

    top1p = jnp.pad(top1, ((0, 0), (0, NSRC - TOPK1)))
    node = jnp.arange(n, dtype=top1p.dtype)
    srcmask = (jnp.arange(NSRC) < TOPK1)
    sel = ((top1p[:, :, None] == node[None, None, :])
           & srcmask[None, :, None]).astype(jnp.float32)
    xs = top1p % w
    xd = node % w
    band = ((jnp.abs(xd[None, :, None] - xs[:, None, :]) <= RAD)
            & srcmask[None, None, :]).astype(jnp.float32)

    h1 = _gat_layer(pf, sel, band, p['Wl1'], p['bl1'], p['Wr1'], p['br1'],
                    p['att1'], p['Wres1'], p['bias1'])
    h2 = _gat_layer(h1, sel, band, p['Wl2'], p['bl2'], p['Wr2'], p['br2'],
                    p['att2'], p['Wres2'], p['bias2'])

    g = jnp.take_along_axis(h2, top1[:, :, None], axis=1)
    scores2 = jnp.linalg.norm(g, axis=2)
    _, top2 = jax.lax.top_k(scores2, TOPK2)
    g = jnp.take_along_axis(g, top2[:, :, None], axis=1)
    g = g.transpose(0, 2, 1)[..., None]
    g = _l2norm(g, 1)
    return _l2norm(_net_vlad(g, p), 1)


# speedup vs baseline: 12.5574x; 1.0165x over previous
import functools

import jax
import jax.numpy as jnp
import numpy as np
from jax.experimental import pallas as pl
from jax.experimental.pallas import tpu as pltpu

TOPK1, TOPK2, RAD = 500, 300, 9
NSRC = 512


def _gat_kernel(xn_ref, s_ref, band_ref, wl_ref, bl_ref, wr_ref, br_ref,
                att_ref, wres_ref, bias_ref, out_ref, e_s, xr_s):
    f32 = jnp.float32
    xn = xn_ref[0]
    xl = jnp.dot(xn, wl_ref[...], preferred_element_type=f32) + bl_ref[...]
    xr = jnp.dot(xn, wr_ref[...], preferred_element_type=f32) + br_ref[...]
    xr_s[...] = xr
    sel = s_ref[0]
    xls = jnp.dot(sel, xl, preferred_element_type=f32)
    att = att_ref[...]

    n = xn.shape[0]
    td = 8

    def body(t, carry):
        xr_t = xr_s[pl.ds(t * td, td), :]
        v = xr_t[:, None, :] + xls[None, :, :]
        v = jnp.where(v >= 0, v, 0.2 * v) * att[None, :, :]
        e_s[pl.ds(t * td, td), :] = jnp.sum(v, axis=2)
        return carry

    jax.lax.fori_loop(0, n // td, body, 0)

    band = band_ref[0]
    e = jnp.where(band > 0.5, e_s[...], -1e9)

    vs = xl + xr
    vs = jnp.where(vs >= 0, vs, 0.2 * vs) * att
    sl = jnp.sum(vs, axis=1, keepdims=True)

    m = jnp.maximum(jnp.max(e, axis=1, keepdims=True), sl)
    ex = jnp.where(band > 0.5, jnp.exp(e - m), 0.0)
    exs = jnp.exp(sl - m)
    den = jnp.sum(ex, axis=1, keepdims=True) + exs
    inv = 1.0 / (den + 1e-16)
    out = (jnp.dot(ex * inv, xls, preferred_element_type=f32)
           + (exs * inv) * xl
           + jnp.dot(xn, wres_ref[...], preferred_element_type=f32)
           + bias_ref[...])
    out_ref[0] = jnp.maximum(out, 0.0)


def _gat_layer(xn, sel, band, wl, bl, wr, br, att, wres, bias):
    b, n, din = xn.shape
    wspec = lambda shp: pl.BlockSpec(shp, lambda i: (0,) * len(shp))
    bspec = lambda shp: pl.BlockSpec(shp, lambda i: (i,) + (0,) * (len(shp) - 1))
    return pl.pallas_call(
        _gat_kernel,
        grid=(b,),
        in_specs=[
            bspec((1, n, din)),
            bspec((1, NSRC, n)),
            bspec((1, n, NSRC)),
            wspec((din, 256)), wspec((1, 256)),
            wspec((din, 256)), wspec((1, 256)),
            wspec((1, 256)),
            wspec((din, 256)), wspec((1, 256)),
        ],
        out_specs=bspec((1, n, 256)),
        out_shape=jax.ShapeDtypeStruct((b, n, 256), jnp.float32),
        scratch_shapes=[pltpu.VMEM((n, NSRC), jnp.float32),
                        pltpu.VMEM((n, 256), jnp.float32)],
    )(xn, sel, band,
      wl.T, bl.reshape(1, 256), wr.T, br.reshape(1, 256),
      att.reshape(1, 256), wres.T, bias.reshape(1, 256))


def _pool_matrix(n, m):
    p = np.zeros((n, m), np.float32)
    for i in range(m):
        s = (i * n) // m
        e = -(-((i + 1) * n) // m)
        p[s:e, i] = 1.0 / (e - s)
    return jnp.asarray(p)


def _conv(x, w, stride, pad):
    return jax.lax.conv_general_dilated(
        x, w, (stride, stride), [(pad, pad), (pad, pad)],
        dimension_numbers=('NCHW', 'OIHW', 'NCHW'))


def _group_norm(x, groups, g, b, eps=1e-5):
    bb, c, h, w = x.shape
    xr = x.reshape(bb, groups, c // groups, h, w)
    m = xr.mean(axis=(2, 3, 4), keepdims=True)
    v = xr.var(axis=(2, 3, 4), keepdims=True)
    x = ((xr - m) / jnp.sqrt(v + eps)).reshape(bb, c, h, w)
    return x * g.reshape(1, c, 1, 1) + b.reshape(1, c, 1, 1)


def _pyramid_cnn(x, p):
    r = jax.nn.relu
    x = r(_group_norm(_conv(x, p['conv1_w'], 1, 2), 4, p['gn1_g'], p['gn1_b']))
    x = r(_group_norm(_conv(x, p['conv1_1_w'], 1, 1), 4, p['gn1_1_g'], p['gn1_1_b']))
    x = r(_group_norm(_conv(x, p['conv2_w'], 2, 2), 8, p['gn2_g'], p['gn2_b']))
    x = r(_group_norm(_conv(x, p['conv2_1_w'], 1, 1), 8, p['gn2_1_g'], p['gn2_1_b']))
    x = r(_group_norm(_conv(x, p['conv3_w'], 2, 2), 16, p['gn3_g'], p['gn3_b']))
    x = r(_group_norm(_conv(x, p['conv3_1_w'], 1, 1), 16, p['gn3_1_g'], p['gn3_1_b']))
    ph = _pool_matrix(x.shape[2], 8)
    pw = _pool_matrix(x.shape[3], 112)
    x = jnp.einsum('bchw,hp,wq->bcpq', x, ph, pw)
    bb, c, h, w = x.shape
    return x.reshape(bb, c, h * w).transpose(0, 2, 1)


def _l2norm(x, axis):
    return x / jnp.clip(jnp.linalg.norm(x, axis=axis, keepdims=True), 1e-12)


def _net_vlad_masked(x, msk, p):
    b = x.shape[0]
    act = jax.nn.softmax(x @ p['cluster_w'] + p['cluster_b'], axis=-1)
    act = act * msk[:, :, None]
    a = act.sum(axis=1, keepdims=True) * p['cluster_w2']
    vlad = jnp.matmul(act.transpose(0, 2, 1), x).transpose(0, 2, 1) - a
    vlad = _l2norm(vlad, 1).reshape(b, -1)
    vlad = _l2norm(vlad, 1) @ p['hidden1_w']
    vlad = vlad * p['bn2_g'] + p['bn2_b']
    gates = jax.nn.sigmoid(vlad @ p['gating_w'] + p['gating_b'])
    return vlad * gates


def _topk_mask(scores, k):
    s_i = scores[:, :, None]
    s_j = scores[:, None, :]
    idx = jnp.arange(scores.shape[1])
    before = (s_j > s_i) | ((s_j == s_i) & (idx[None, None, :] < idx[None, :, None]))
    rank = jnp.sum(before.astype(jnp.float32), axis=2)
    return (rank < k).astype(jnp.float32)


def kernel(x, params):
    p = params
    pf = _pyramid_cnn(x, p)
    h, w = x.shape[2] // 8, x.shape[3] // 8
    n = h * w

    scores1 = jnp.linalg.norm(pf, axis=2)
    mask1 = _topk_mask(scores1, TOPK1)
    slot = jnp.cumsum(mask1, axis=1) - 1.0
    srange = jnp.arange(NSRC, dtype=jnp.float32)
    sel = ((slot[:, None, :] == srange[None, :, None])
           & (mask1[:, None, :] > 0.5)).astype(jnp.float32)
    xd = (jnp.arange(n) % w).astype(jnp.float32)
    xs = jnp.einsum('bsn,n->bs', sel, xd)
    srcvalid = jnp.arange(NSRC) < TOPK1
    band = ((jnp.abs(xd[None, :, None] - xs[:, None, :]) <= RAD)
            & srcvalid[None, None, :]).astype(jnp.float32)

    h1 = _gat_layer(pf, sel, band, p['Wl1'], p['bl1'], p['Wr1'], p['br1'],
                    p['att1'], p['Wres1'], p['bias1'])
    h2 = _gat_layer(h1, sel, band, p['Wl2'], p['bl2'], p['Wr2'], p['br2'],
                    p['att2'], p['Wres2'], p['bias2'])

    g = jnp.einsum('bsn,bnc->bsc', sel, h2)
    scores2 = jnp.linalg.norm(g, axis=2)
    scores2 = jnp.where(srcvalid[None, :], scores2, -1.0)
    mask2 = _topk_mask(scores2, TOPK2)
    g = _l2norm(g, 2)
    return _l2norm(_net_vlad_masked(g, mask2, p), 1)


# speedup vs baseline: 14.2393x; 1.1339x over previous
import functools

import jax
import jax.numpy as jnp
import numpy as np
from jax.experimental import pallas as pl
from jax.experimental.pallas import tpu as pltpu

TOPK1, TOPK2, RAD = 500, 300, 9
NSRC = 512


def _gat_kernel(xn_ref, s_ref, band_ref, wl_ref, bl_ref, wr_ref, br_ref,
                att_ref, wres_ref, bias_ref, out_ref, e_s, xr_s):
    f32 = jnp.float32
    xn = xn_ref[0]
    xl = jnp.dot(xn, wl_ref[...], preferred_element_type=f32) + bl_ref[...]
    xr = jnp.dot(xn, wr_ref[...], preferred_element_type=f32) + br_ref[...]
    xr_s[...] = xr
    sel = s_ref[0]
    xls = jnp.dot(sel, xl, preferred_element_type=f32)
    att = att_ref[...]

    n = xn.shape[0]
    td = 8

    def body(t, carry):
        xr_t = xr_s[pl.ds(t * td, td), :]
        v = xr_t[:, None, :] + xls[None, :, :]
        v = jnp.where(v >= 0, v, 0.2 * v) * att[None, :, :]
        e_s[pl.ds(t * td, td), :] = jnp.sum(v, axis=2)
        return carry

    jax.lax.fori_loop(0, n // td, body, 0)

    band = band_ref[0]
    e = jnp.where(band > 0.5, e_s[...], -1e9)

    vs = xl + xr
    vs = jnp.where(vs >= 0, vs, 0.2 * vs) * att
    sl = jnp.sum(vs, axis=1, keepdims=True)

    m = jnp.maximum(jnp.max(e, axis=1, keepdims=True), sl)
    ex = jnp.where(band > 0.5, jnp.exp(e - m), 0.0)
    exs = jnp.exp(sl - m)
    den = jnp.sum(ex, axis=1, keepdims=True) + exs
    inv = 1.0 / (den + 1e-16)
    out = (jnp.dot(ex * inv, xls, preferred_element_type=f32)
           + (exs * inv) * xl
           + jnp.dot(xn, wres_ref[...], preferred_element_type=f32)
           + bias_ref[...])
    out_ref[0] = jnp.maximum(out, 0.0)


def _gat_layer(xn, sel, band, wl, bl, wr, br, att, wres, bias):
    b, n, din = xn.shape
    wspec = lambda shp: pl.BlockSpec(shp, lambda i: (0,) * len(shp))
    bspec = lambda shp: pl.BlockSpec(shp, lambda i: (i,) + (0,) * (len(shp) - 1))
    return pl.pallas_call(
        _gat_kernel,
        grid=(b,),
        in_specs=[
            bspec((1, n, din)),
            bspec((1, NSRC, n)),
            bspec((1, n, NSRC)),
            wspec((din, 256)), wspec((1, 256)),
            wspec((din, 256)), wspec((1, 256)),
            wspec((1, 256)),
            wspec((din, 256)), wspec((1, 256)),
        ],
        out_specs=bspec((1, n, 256)),
        out_shape=jax.ShapeDtypeStruct((b, n, 256), jnp.float32),
        scratch_shapes=[pltpu.VMEM((n, NSRC), jnp.float32),
                        pltpu.VMEM((n, 256), jnp.float32)],
    )(xn, sel, band,
      wl.T, bl.reshape(1, 256), wr.T, br.reshape(1, 256),
      att.reshape(1, 256), wres.T, bias.reshape(1, 256))


def _pool_matrix(n, m):
    p = np.zeros((n, m), np.float32)
    for i in range(m):
        s = (i * n) // m
        e = -(-((i + 1) * n) // m)
        p[s:e, i] = 1.0 / (e - s)
    return jnp.asarray(p)


def _conv(x, w, stride, pad):
    return jax.lax.conv_general_dilated(
        x, w.transpose(2, 3, 1, 0), (stride, stride), [(pad, pad), (pad, pad)],
        dimension_numbers=('NHWC', 'HWIO', 'NHWC'))


def _group_norm(x, groups, g, b, eps=1e-5):
    bb, h, w, c = x.shape
    xr = x.reshape(bb, h, w, groups, c // groups)
    m = xr.mean(axis=(1, 2, 4), keepdims=True)
    v = xr.var(axis=(1, 2, 4), keepdims=True)
    x = ((xr - m) / jnp.sqrt(v + eps)).reshape(bb, h, w, c)
    return x * g.reshape(1, 1, 1, c) + b.reshape(1, 1, 1, c)


def _gn_apply(am, xokf, cgrp_cnt, c, g_row, b_row, eps=1e-5):
    s1 = jnp.sum(am, axis=0, keepdims=True)
    s2 = jnp.sum(am * am, axis=0, keepdims=True)
    gi = jax.lax.broadcasted_iota(jnp.int32, (c, c), 0) >> 3
    gj = jax.lax.broadcasted_iota(jnp.int32, (c, c), 1) >> 3
    grp = (gi == gj).astype(jnp.float32)
    s1g = jnp.dot(s1, grp, preferred_element_type=jnp.float32)
    s2g = jnp.dot(s2, grp, preferred_element_type=jnp.float32)
    m = s1g / cgrp_cnt
    v = s2g / cgrp_cnt - m * m
    rs = 1.0 / jnp.sqrt(v + eps)
    out = (am - m) * rs * g_row + b_row
    return jnp.maximum(out, 0.0) * xokf


def _xok(mrows, wp, w):
    xi = jax.lax.broadcasted_iota(jnp.int32, (mrows, 1), 0) & (wp - 1)
    return (xi < w).astype(jnp.float32)


def _taps(r):
    return [(dy, dx) for dy in range(-r, r + 1) for dx in range(-r, r + 1)]


def _cnn_bc_kernel(pk_ref, w5_ref, g1_ref, b1_ref,
                   w3_ref, g2_ref, b2_ref, o_ref, pscr,
                   *, wp, h, w, cin, cout):
    m = h * wp
    first = True
    for ey in range(-2, 3):
        for ex in range(-2, 3):
            pc = (ey & 1) * 2 + (ex & 1)
            sy, sx = ey >> 1, ex >> 1
            ofs = 2 * wp + sy * wp + sx
            t = (ey + 2) * 5 + (ex + 2)
            part = jnp.dot(pk_ref[0, pl.ds(ofs, m), pl.ds(pc * cin, cin)],
                           w5_ref[pl.ds(t * cin, cin), :],
                           preferred_element_type=jnp.float32)
            if first:
                o_ref[0] = part
                first = False
            else:
                o_ref[0] += part
    xokf = _xok(m, wp, w)
    am = o_ref[0] * xokf
    mid = _gn_apply(am, xokf, float(h * w * 8), cout, g1_ref[...], b1_ref[...])
    pscr[...] = jnp.zeros_like(pscr)
    pscr[pl.ds(2 * wp, m), :] = mid
    for t, (dy, dx) in enumerate(_taps(1)):
        ofs = 2 * wp + dy * wp + dx
        part = jnp.dot(pscr[pl.ds(ofs, m), :], w3_ref[pl.ds(t * cout, cout), :],
                       preferred_element_type=jnp.float32)
        if t == 0:
            o_ref[0] = part
        else:
            o_ref[0] += part
    am2 = o_ref[0] * xokf
    o_ref[0] = _gn_apply(am2, xokf, float(h * w * 8), cout, g2_ref[...], b2_ref[...])


def _row_spec(shp):
    return pl.BlockSpec(shp, lambda i: (i,) + (0,) * (len(shp) - 1))


def _full_spec(shp):
    return pl.BlockSpec(shp, lambda i: (0,) * len(shp))


def _phases(a, b_, h, wp_in, c, w_out, wp_out):
    outs = []
    for yp in (0, 1):
        for xp in (0, 1):
            ph = a[:, yp::2, xp::2, :][:, :, :w_out, :]
            ph = jnp.pad(ph, ((0, 0), (2, 2), (0, wp_out - w_out), (0, 0)))
            outs.append(ph.reshape(b_, (h // 2 + 4) * wp_out, c))
    return jnp.concatenate(outs, axis=2)


def _pyramid_cnn(x, p):
    r = jax.nn.relu
    b_ = x.shape[0]
    x = x.transpose(0, 2, 3, 1)
    a1 = r(_group_norm(_conv(x, p['conv1_w'], 1, 2), 4, p['gn1_g'], p['gn1_b']))
    a2 = r(_group_norm(_conv(a1, p['conv1_1_w'], 1, 1), 4,
                       p['gn1_1_g'], p['gn1_1_b']))

    ph2 = _phases(a2, b_, 64, 1024, 32, 450, 512)
    w2 = p['conv2_w'].transpose(2, 3, 1, 0).reshape(25 * 32, 64)
    w21 = p['conv2_1_w'].transpose(2, 3, 1, 0).reshape(9 * 64, 64)
    kb = functools.partial(_cnn_bc_kernel, wp=512, h=32, w=450, cin=32, cout=64)
    a3 = pl.pallas_call(
        kb,
        grid=(b_,),
        in_specs=[_row_spec((1, 36 * 512, 128)),
                  _full_spec((25 * 32, 64)), _full_spec((1, 64)),
                  _full_spec((1, 64)), _full_spec((9 * 64, 64)),
                  _full_spec((1, 64)), _full_spec((1, 64))],
        out_specs=_row_spec((1, 32 * 512, 64)),
        out_shape=jax.ShapeDtypeStruct((b_, 32 * 512, 64), jnp.float32),
        scratch_shapes=[pltpu.VMEM((36 * 512, 64), jnp.float32)],
    )(ph2, w2, p['gn2_g'].reshape(1, 64), p['gn2_b'].reshape(1, 64),
      w21, p['gn2_1_g'].reshape(1, 64), p['gn2_1_b'].reshape(1, 64))

    ph3 = _phases(a3.reshape(b_, 32, 512, 64), b_, 32, 512, 64, 225, 256)
    w3 = p['conv3_w'].transpose(2, 3, 1, 0).reshape(25 * 64, 128)
    w31 = p['conv3_1_w'].transpose(2, 3, 1, 0).reshape(9 * 128, 128)
    kc = functools.partial(_cnn_bc_kernel, wp=256, h=16, w=225, cin=64, cout=128)
    a4 = pl.pallas_call(
        kc,
        grid=(b_,),
        in_specs=[_row_spec((1, 20 * 256, 256)),
                  _full_spec((25 * 64, 128)), _full_spec((1, 128)),
                  _full_spec((1, 128)), _full_spec((9 * 128, 128)),
                  _full_spec((1, 128)), _full_spec((1, 128))],
        out_specs=_row_spec((1, 16 * 256, 128)),
        out_shape=jax.ShapeDtypeStruct((b_, 16 * 256, 128), jnp.float32),
        scratch_shapes=[pltpu.VMEM((20 * 256, 128), jnp.float32)],
    )(ph3, w3, p['gn3_g'].reshape(1, 128), p['gn3_b'].reshape(1, 128),
      w31, p['gn3_1_g'].reshape(1, 128), p['gn3_1_b'].reshape(1, 128))

    a4 = a4.reshape(b_, 16, 256, 128)[:, :, :225, :]
    ph_m = _pool_matrix(16, 8)
    pw_m = _pool_matrix(225, 112)
    a4 = jnp.einsum('bhwc,hp,wq->bpqc', a4, ph_m, pw_m)
    return a4.reshape(b_, 896, 128)


def _l2norm(x, axis):
    return x / jnp.clip(jnp.linalg.norm(x, axis=axis, keepdims=True), 1e-12)


def _net_vlad_masked(x, msk, p):
    b = x.shape[0]
    act = jax.nn.softmax(x @ p['cluster_w'] + p['cluster_b'], axis=-1)
    act = act * msk[:, :, None]
    a = act.sum(axis=1, keepdims=True) * p['cluster_w2']
    vlad = jnp.matmul(act.transpose(0, 2, 1), x).transpose(0, 2, 1) - a
    vlad = _l2norm(vlad, 1).reshape(b, -1)
    vlad = _l2norm(vlad, 1) @ p['hidden1_w']
    vlad = vlad * p['bn2_g'] + p['bn2_b']
    gates = jax.nn.sigmoid(vlad @ p['gating_w'] + p['gating_b'])
    return vlad * gates


def _topk_mask(scores, k):
    s_i = scores[:, :, None]
    s_j = scores[:, None, :]
    idx = jnp.arange(scores.shape[1])
    before = (s_j > s_i) | ((s_j == s_i) & (idx[None, None, :] < idx[None, :, None]))
    rank = jnp.sum(before.astype(jnp.float32), axis=2)
    return (rank < k).astype(jnp.float32)


def kernel(x, params):
    p = params
    pf = _pyramid_cnn(x, p)
    h, w = x.shape[2] // 8, x.shape[3] // 8
    n = h * w

    scores1 = jnp.linalg.norm(pf, axis=2)
    mask1 = _topk_mask(scores1, TOPK1)
    slot = jnp.cumsum(mask1, axis=1) - 1.0
    srange = jnp.arange(NSRC, dtype=jnp.float32)
    sel = ((slot[:, None, :] == srange[None, :, None])
           & (mask1[:, None, :] > 0.5)).astype(jnp.float32)
    xd = (jnp.arange(n) % w).astype(jnp.float32)
    xs = jnp.einsum('bsn,n->bs', sel, xd)
    srcvalid = jnp.arange(NSRC) < TOPK1
    band = ((jnp.abs(xd[None, :, None] - xs[:, None, :]) <= RAD)
            & srcvalid[None, None, :]).astype(jnp.float32)

    h1 = _gat_layer(pf, sel, band, p['Wl1'], p['bl1'], p['Wr1'], p['br1'],
                    p['att1'], p['Wres1'], p['bias1'])
    h2 = _gat_layer(h1, sel, band, p['Wl2'], p['bl2'], p['Wr2'], p['br2'],
                    p['att2'], p['Wres2'], p['bias2'])

    g = jnp.einsum('bsn,bnc->bsc', sel, h2)
    scores2 = jnp.linalg.norm(g, axis=2)
    scores2 = jnp.where(srcvalid[None, :], scores2, -1.0)
    mask2 = _topk_mask(scores2, TOPK2)
    g = _l2norm(g, 2)
    return _l2norm(_net_vlad_masked(g, mask2, p), 1)
